# baseline (speedup 1.0000x reference)
import functools

import jax
import jax.numpy as jnp
from jax import lax
from jax.experimental import pallas as pl
from jax.experimental.pallas import tpu as pltpu
from jax.experimental.pallas import tpu_sc as plsc

N_ITEMS = 100000
TEXT_DIM = 384
IMAGE_DIM = 512
OUT_DIM = TEXT_DIM + IMAGE_DIM
BATCH = 16384

_info = plsc.get_sparse_core_info()
_NC, _NS = _info.num_cores, _info.num_subcores
_NW = _NC * _NS
_B_PER_W = BATCH // _NW
_CHUNK = 32
_NBUF = 4
_N_CHUNKS = _B_PER_W // _CHUNK
_AHEAD = 2


def _sc_gather_concat(idx_hbm, text_hbm, image_hbm, out_hbm,
                      idx_v, b0, b1, b2, b3,
                      g0, g1, g2, g3, w0, w1, w2, w3):
    wid = lax.axis_index("s") * _NC + lax.axis_index("c")
    base = wid * _B_PER_W
    pltpu.sync_copy(idx_hbm.at[pl.ds(base, _B_PER_W)], idx_v)
    bufs = (b0, b1, b2, b3)
    gsems = (g0, g1, g2, g3)
    wsems = (w0, w1, w2, w3)
    gh = {}
    wh = {}

    def fire_gather(c):
        b = c % _NBUF
        idx_chunk = idx_v.at[pl.ds(c * _CHUNK, _CHUNK)]
        gh[b] = (
            pltpu.async_copy(
                text_hbm.at[idx_chunk], bufs[b].at[:, pl.ds(0, TEXT_DIM)],
                gsems[b]),
            pltpu.async_copy(
                image_hbm.at[idx_chunk],
                bufs[b].at[:, pl.ds(TEXT_DIM, IMAGE_DIM)], gsems[b]),
        )

    for c in range(_AHEAD):
        fire_gather(c)
    for c in range(_N_CHUNKS):
        b = c % _NBUF
        ht, hi = gh[b]
        ht.wait()
        hi.wait()
        row0 = base + c * _CHUNK
        wh[b] = pltpu.async_copy(
            bufs[b], out_hbm.at[pl.ds(row0, _CHUNK)], wsems[b])
        n = c + _AHEAD
        if n < _N_CHUNKS:
            bn = n % _NBUF
            if n >= _NBUF:
                wh[bn].wait()
            fire_gather(n)
    for b in range(_NBUF):
        wh[b].wait()


@jax.jit
def _encode(item_idx, text_features, image_features):
    mesh = plsc.VectorSubcoreMesh(core_axis_name="c", subcore_axis_name="s")
    run = functools.partial(
        pl.kernel,
        mesh=mesh,
        out_type=jax.ShapeDtypeStruct((BATCH, OUT_DIM), jnp.float32),
        scratch_types=(
            [pltpu.VMEM((_B_PER_W,), jnp.int32)]
            + [pltpu.VMEM((_CHUNK, OUT_DIM), jnp.float32)] * _NBUF
            + [pltpu.SemaphoreType.DMA] * (2 * _NBUF)
        ),
    )(_sc_gather_concat)
    return run(item_idx.astype(jnp.int32), text_features, image_features)


def kernel(item_idx, text_features, image_features):
    return _encode(item_idx, text_features, image_features)
